# Initial kernel scaffold; baseline (speedup 1.0000x reference)
import functools

import jax
import jax.numpy as jnp
from jax import lax
from jax.experimental import pallas as pl
from jax.experimental.pallas import tpu as pltpu
from jax.experimental.pallas import tpu_sc as plsc

_B = 16384
_C = 20
_NEG = 20
_D = 64
_K = _C + _NEG
_NC = 2
_NS = 16
_NW = _NC * _NS
_L = 16
_DV = _D // _L
_RPW = _B // _NW
_RCH = 16
_NCH = _RPW // _RCH
_GPC = _RCH * _K // 128


def _sc_gather_reduce(idx2d, t, W_in, W_out):
  mesh = plsc.VectorSubcoreMesh(
      core_axis_name="c", subcore_axis_name="s",
      num_cores=_NC, num_subcores=_NS)

  @functools.partial(
      pl.kernel,
      out_type=(jax.ShapeDtypeStruct((_B, _L), jnp.float32),
                jax.ShapeDtypeStruct((_B, _L), jnp.float32)),
      mesh=mesh,
      scratch_types=[
          pltpu.VMEM((_GPC, 128), jnp.int32),
          pltpu.VMEM((_RCH,), jnp.int32),
          pltpu.VMEM((_RCH * _K, _D), jnp.float32),
          pltpu.VMEM((_RCH, _D), jnp.float32),
          pltpu.VMEM((_RCH, _L), jnp.float32),
          pltpu.VMEM((_RCH, _L), jnp.float32),
          pltpu.SemaphoreType.DMA,
      ],
  )
  def sc_k(idx_hbm, t_hbm, win_hbm, wout_hbm, pos_hbm, neg_hbm,
           idx_v, t_v, rows_v, vrow_v, posb, negb, sem):
    wid = lax.axis_index("s") * _NC + lax.axis_index("c")

    def chunk_body(c, carry):
      row0 = wid * _RPW + c * _RCH
      irow0 = wid * (_RPW * _K // 128) + c * _GPC
      pltpu.sync_copy(idx_hbm.at[pl.ds(irow0, _GPC)], idx_v)
      pltpu.sync_copy(t_hbm.at[pl.ds(row0, _RCH)], t_v)
      cps = [pltpu.async_copy(wout_hbm.at[idx_v.at[j]],
                              rows_v.at[pl.ds(j * 128, 128)], sem)
             for j in range(_GPC)]
      cps.append(pltpu.async_copy(win_hbm.at[t_v], vrow_v, sem))
      for cp in cps:
        cp.wait()

      def row_body(r, rcarry):
        base = r * _K
        accp = [rows_v[base, pl.ds(k * _L, _L)] for k in range(_DV)]
        for j in range(1, _C):
          for k in range(_DV):
            accp[k] = accp[k] + rows_v[base + j, pl.ds(k * _L, _L)]
        accn = [rows_v[base + _C, pl.ds(k * _L, _L)] for k in range(_DV)]
        for j in range(_C + 1, _K):
          for k in range(_DV):
            accn[k] = accn[k] + rows_v[base + j, pl.ds(k * _L, _L)]
        wp = jnp.zeros((_L,), jnp.float32)
        wn = jnp.zeros((_L,), jnp.float32)
        for k in range(_DV):
          vk = vrow_v[r, pl.ds(k * _L, _L)]
          wp = wp + accp[k] * vk
          wn = wn + accn[k] * vk
        posb[r, :] = wp
        negb[r, :] = wn
        return rcarry

      lax.fori_loop(0, _RCH, row_body, 0)
      pltpu.sync_copy(posb, pos_hbm.at[pl.ds(row0, _RCH)])
      pltpu.sync_copy(negb, neg_hbm.at[pl.ds(row0, _RCH)])
      return carry

    lax.fori_loop(0, _NCH, chunk_body, 0)

  return sc_k(idx2d, t, W_in, W_out)


def _tc_loss(pos_part, neg_part):
  def body(p_ref, n_ref, o_ref):
    ps = jnp.sum(p_ref[...], axis=1)
    ns = jnp.sum(n_ref[...], axis=1)
    lp = jnp.minimum(ps, 0.0) - jnp.log(1.0 + jnp.exp(-jnp.abs(ps)))
    x = -ns
    ln = jnp.minimum(x, 0.0) - jnp.log(1.0 + jnp.exp(-jnp.abs(x)))
    o_ref[0, 0] = -(jnp.sum(lp + ln) / _B)

  return pl.pallas_call(
      body,
      out_shape=jax.ShapeDtypeStruct((1, 1), jnp.float32),
      out_specs=pl.BlockSpec(memory_space=pltpu.SMEM),
  )(pos_part, neg_part)


def kernel(target_input, context, neg, W_in, W_out):
  t = target_input.reshape(_B)
  idx2d = jnp.concatenate([context, neg], axis=1).reshape(_B * _K // 128, 128)
  pos_part, neg_part = _sc_gather_reduce(idx2d, t, W_in, W_out)
  return _tc_loss(pos_part, neg_part).reshape(())


# speedup vs baseline: 8.6488x; 8.6488x over previous
import functools

import jax
import jax.numpy as jnp
from jax import lax
from jax.experimental import pallas as pl
from jax.experimental.pallas import tpu as pltpu
from jax.experimental.pallas import tpu_sc as plsc

_B = 16384
_C = 20
_NEG = 20
_D = 64
_K = _C + _NEG
_NC = 2
_NS = 16
_NW = _NC * _NS
_L = 16
_DV = _D // _L
_RPW = _B // _NW
_RCH = 16
_NCH = _RPW // _RCH
_GPC = _RCH * _K // 128


def _sc_gather_reduce(idx2d, t, W_in, W_out):
  mesh = plsc.VectorSubcoreMesh(
      core_axis_name="c", subcore_axis_name="s",
      num_cores=_NC, num_subcores=_NS)

  @functools.partial(
      pl.kernel,
      out_type=(jax.ShapeDtypeStruct((_B, _L), jnp.float32),
                jax.ShapeDtypeStruct((_B, _L), jnp.float32)),
      mesh=mesh,
      compiler_params=pltpu.CompilerParams(use_tc_tiling_on_sc=False),
      scratch_types=[
          pltpu.VMEM((_RCH * _K,), jnp.int32),
          pltpu.VMEM((_RCH,), jnp.int32),
          pltpu.VMEM((_RCH * _K, _D), jnp.float32),
          pltpu.VMEM((_RCH, _D), jnp.float32),
          pltpu.VMEM((_RCH, _L), jnp.float32),
          pltpu.VMEM((_RCH, _L), jnp.float32),
          pltpu.SemaphoreType.DMA,
      ],
  )
  def sc_k(idx_hbm, t_hbm, win_hbm, wout_hbm, pos_hbm, neg_hbm,
           idx_v, t_v, rows_v, vrow_v, posb, negb, sem):
    wid = lax.axis_index("s") * _NC + lax.axis_index("c")

    def chunk_body(c, carry):
      row0 = wid * _RPW + c * _RCH
      ibase = row0 * _K
      pltpu.sync_copy(idx_hbm.at[pl.ds(ibase, _RCH * _K)], idx_v)
      pltpu.sync_copy(t_hbm.at[pl.ds(row0, _RCH)], t_v)
      cps = [pltpu.async_copy(wout_hbm.at[idx_v.at[pl.ds(j * 128, 128)]],
                              rows_v.at[pl.ds(j * 128, 128)], sem)
             for j in range(_GPC)]
      cps.append(pltpu.async_copy(win_hbm.at[t_v], vrow_v, sem))
      for cp in cps:
        cp.wait()

      def row_body(r, rcarry):
        base = r * _K
        accp = [rows_v[base, pl.ds(k * _L, _L)] for k in range(_DV)]
        for j in range(1, _C):
          for k in range(_DV):
            accp[k] = accp[k] + rows_v[base + j, pl.ds(k * _L, _L)]
        accn = [rows_v[base + _C, pl.ds(k * _L, _L)] for k in range(_DV)]
        for j in range(_C + 1, _K):
          for k in range(_DV):
            accn[k] = accn[k] + rows_v[base + j, pl.ds(k * _L, _L)]
        wp = jnp.zeros((_L,), jnp.float32)
        wn = jnp.zeros((_L,), jnp.float32)
        for k in range(_DV):
          vk = vrow_v[r, pl.ds(k * _L, _L)]
          wp = wp + accp[k] * vk
          wn = wn + accn[k] * vk
        posb[r, :] = wp
        negb[r, :] = wn
        return rcarry

      lax.fori_loop(0, _RCH, row_body, 0)
      pltpu.sync_copy(posb, pos_hbm.at[pl.ds(row0, _RCH)])
      pltpu.sync_copy(negb, neg_hbm.at[pl.ds(row0, _RCH)])
      return carry

    lax.fori_loop(0, _NCH, chunk_body, 0)

  return sc_k(idx2d, t, W_in, W_out)


def _tc_loss(pos_part, neg_part):
  def body(p_ref, n_ref, o_ref):
    ps = jnp.sum(p_ref[...], axis=1)
    ns = jnp.sum(n_ref[...], axis=1)
    lp = jnp.minimum(ps, 0.0) - jnp.log(1.0 + jnp.exp(-jnp.abs(ps)))
    x = -ns
    ln = jnp.minimum(x, 0.0) - jnp.log(1.0 + jnp.exp(-jnp.abs(x)))
    o_ref[0, 0] = -(jnp.sum(lp + ln) / _B)

  return pl.pallas_call(
      body,
      out_shape=jax.ShapeDtypeStruct((1, 1), jnp.float32),
      out_specs=pl.BlockSpec(memory_space=pltpu.SMEM),
  )(pos_part, neg_part)


def kernel(target_input, context, neg, W_in, W_out):
  t = target_input.reshape(_B)
  idx1d = jnp.concatenate([context, neg], axis=1).reshape(_B * _K)
  pos_part, neg_part = _sc_gather_reduce(idx1d, t, W_in, W_out)
  return _tc_loss(pos_part, neg_part).reshape(())
